# Initial kernel scaffold; baseline (speedup 1.0000x reference)
import jax
import jax.numpy as jnp
from jax.experimental import pallas as pl


def kernel(gene_ids, expr_values, emb_table, proj_w, proj_b):
    raise NotImplementedError("write your pallas kernel here")


# speedup vs baseline: 4.1807x; 4.1807x over previous
import functools

import jax
import jax.numpy as jnp
from jax import lax
from jax.experimental import pallas as pl
from jax.experimental.pallas import tpu as pltpu
from jax.experimental.pallas import tpu_sc as plsc

NC = 2
NS = 16
NW = NC * NS
L = 16

D = 64
CHUNK = 128
NBUF = 4
UNROLL = 4

TOTAL = 4096 * 200
PER_W = TOTAL // NW
NCHUNK = PER_W // CHUNK

_mesh = plsc.VectorSubcoreMesh(core_axis_name="c", subcore_axis_name="s")


@functools.partial(
    pl.kernel,
    mesh=_mesh,
    compiler_params=pltpu.CompilerParams(use_tc_tiling_on_sc=False),
    out_type=jax.ShapeDtypeStruct((TOTAL, D), jnp.float32),
    scratch_types=[
        pltpu.VMEM((NCHUNK, CHUNK), jnp.int32),
        pltpu.VMEM((PER_W,), jnp.float32),
        pltpu.VMEM((D,), jnp.float32),
        pltpu.VMEM((D,), jnp.float32),
        pltpu.VMEM((CHUNK, D), jnp.float32),
        pltpu.VMEM((CHUNK, D), jnp.float32),
        pltpu.VMEM((CHUNK, D), jnp.float32),
        pltpu.VMEM((CHUNK, D), jnp.float32),
        pltpu.SemaphoreType.DMA,
        pltpu.SemaphoreType.DMA,
        pltpu.SemaphoreType.DMA,
        pltpu.SemaphoreType.DMA,
        pltpu.SemaphoreType.DMA,
        pltpu.SemaphoreType.DMA,
        pltpu.SemaphoreType.DMA,
        pltpu.SemaphoreType.DMA,
    ],
)
def _sc_tokenize(idx_hbm, expr_hbm, table_hbm, w_hbm, b_hbm, out_hbm,
                 idx_v, expr_v, w_v, b_v,
                 r0, r1, r2, r3,
                 g0, g1, g2, g3, s0, s1, s2, s3):
    rows = [r0, r1, r2, r3]
    gsem = [g0, g1, g2, g3]
    ssem = [s0, s1, s2, s3]

    wid = lax.axis_index("s") * NC + lax.axis_index("c")

    pltpu.sync_copy(idx_hbm.at[wid], idx_v)
    pltpu.sync_copy(expr_hbm.at[wid], expr_v)
    pltpu.sync_copy(w_hbm, w_v)
    pltpu.sync_copy(b_hbm, b_v)

    wq = [w_v[pl.ds(q * L, L)] for q in range(4)]
    bq = [b_v[pl.ds(q * L, L)] for q in range(4)]

    out_base = wid * PER_W

    def start_gather(c, s):
        pltpu.make_async_copy(table_hbm.at[idx_v.at[c]], rows[s], gsem[s]).start()

    def wait_gather(s):
        pltpu.make_async_copy(table_hbm.at[idx_v.at[0]], rows[s], gsem[s]).wait()

    def start_store(c, s):
        dst = out_hbm.at[pl.ds(out_base + c * CHUNK, CHUNK)]
        pltpu.make_async_copy(rows[s], dst, ssem[s]).start()

    def wait_store(s):
        dst = out_hbm.at[pl.ds(out_base, CHUNK)]
        pltpu.make_async_copy(rows[s], dst, ssem[s]).wait()

    def compute(c, s):
        r = rows[s]
        ebase = c * CHUNK

        def body(i, carry):
            ev16 = expr_v[pl.ds(ebase + i * L, L)]
            for u in range(L):
                ii = i * L + u
                ev = jnp.full((L,), ev16[u], dtype=jnp.float32)
                for q in range(4):
                    sl = pl.ds(q * L, L)
                    r[ii, sl] = r[ii, sl] + (ev * wq[q] + bq[q])
            return carry

        lax.fori_loop(0, CHUNK // L, body, 0)

    def iteration(j, slot, do_wait_store, do_gather):
        s2 = (slot + 2) % NBUF
        if do_wait_store:
            wait_store(s2)
        if do_gather:
            start_gather(j + 2, s2)
        wait_gather(slot)
        compute(j, slot)
        start_store(j, slot)

    start_gather(0, 0)
    start_gather(1, 1)

    iteration(0, 0, False, True)
    iteration(1, 1, False, True)

    def quad(qi, carry):
        jbase = 2 + 4 * qi
        for bpos in range(4):
            iteration(jbase + bpos, (2 + bpos) % NBUF, True, True)
        return carry

    lax.fori_loop(0, (NCHUNK - 4) // NBUF, quad, 0)

    iteration(NCHUNK - 2, (NCHUNK - 2) % NBUF, True, False)
    iteration(NCHUNK - 1, (NCHUNK - 1) % NBUF, True, False)

    wait_store((NCHUNK - 2) % NBUF)
    wait_store((NCHUNK - 1) % NBUF)


def kernel(gene_ids, expr_values, emb_table, proj_w, proj_b):
    B, S = gene_ids.shape
    idx = gene_ids.reshape(NW, NCHUNK, CHUNK).astype(jnp.int32)
    expr = expr_values.reshape(NW, PER_W).astype(jnp.float32)
    w = proj_w.reshape(D).astype(jnp.float32)
    b = proj_b.reshape(D).astype(jnp.float32)
    out = _sc_tokenize(idx, expr, emb_table, w, b)
    return out.reshape(B, S, D)


# speedup vs baseline: 4.2011x; 1.0049x over previous
import functools

import jax
import jax.numpy as jnp
from jax import lax
from jax.experimental import pallas as pl
from jax.experimental.pallas import tpu as pltpu
from jax.experimental.pallas import tpu_sc as plsc

NC = 2
NS = 16
NW = NC * NS
L = 16

D = 64
B = 4096
S = 200
RPW = B // NW
NBUF = 4
SPLIT = 104

_mesh = plsc.VectorSubcoreMesh(core_axis_name="c", subcore_axis_name="s")


@functools.partial(
    pl.kernel,
    mesh=_mesh,
    compiler_params=pltpu.CompilerParams(use_tc_tiling_on_sc=False),
    out_type=jax.ShapeDtypeStruct((B, S, D), jnp.float32),
    scratch_types=[
        pltpu.VMEM((RPW, S), jnp.int32),
        pltpu.VMEM((RPW, S), jnp.float32),
        pltpu.VMEM((D,), jnp.float32),
        pltpu.VMEM((D,), jnp.float32),
        pltpu.VMEM((S, D), jnp.float32),
        pltpu.VMEM((S, D), jnp.float32),
        pltpu.VMEM((S, D), jnp.float32),
        pltpu.VMEM((S, D), jnp.float32),
        pltpu.SemaphoreType.DMA,
        pltpu.SemaphoreType.DMA,
        pltpu.SemaphoreType.DMA,
        pltpu.SemaphoreType.DMA,
        pltpu.SemaphoreType.DMA,
        pltpu.SemaphoreType.DMA,
        pltpu.SemaphoreType.DMA,
        pltpu.SemaphoreType.DMA,
    ],
)
def _sc_tokenize(idx_hbm, expr_hbm, table_hbm, w_hbm, b_hbm, out_hbm,
                 idx_v, expr_v, w_v, b_v,
                 r0, r1, r2, r3,
                 g0, g1, g2, g3, s0, s1, s2, s3):
    rows = [r0, r1, r2, r3]
    gsem = [g0, g1, g2, g3]
    ssem = [s0, s1, s2, s3]

    wid = lax.axis_index("s") * NC + lax.axis_index("c")
    row_base = wid * RPW

    pltpu.sync_copy(idx_hbm.at[pl.ds(row_base, RPW)], idx_v)
    pltpu.sync_copy(expr_hbm.at[pl.ds(row_base, RPW)], expr_v)
    pltpu.sync_copy(w_hbm, w_v)
    pltpu.sync_copy(b_hbm, b_v)

    wq = [w_v[pl.ds(q * L, L)] for q in range(4)]
    bq = [b_v[pl.ds(q * L, L)] for q in range(4)]

    def start_gather(c, s):
        pltpu.make_async_copy(
            table_hbm.at[idx_v.at[c, pl.ds(0, SPLIT)]],
            rows[s].at[pl.ds(0, SPLIT)], gsem[s]).start()
        pltpu.make_async_copy(
            table_hbm.at[idx_v.at[c, pl.ds(SPLIT, S - SPLIT)]],
            rows[s].at[pl.ds(SPLIT, S - SPLIT)], gsem[s]).start()

    def wait_gather(s):
        pltpu.make_async_copy(
            table_hbm.at[idx_v.at[0, pl.ds(0, SPLIT)]],
            rows[s].at[pl.ds(0, SPLIT)], gsem[s]).wait()
        pltpu.make_async_copy(
            table_hbm.at[idx_v.at[0, pl.ds(SPLIT, S - SPLIT)]],
            rows[s].at[pl.ds(SPLIT, S - SPLIT)], gsem[s]).wait()

    def start_store(c, s):
        pltpu.make_async_copy(rows[s], out_hbm.at[row_base + c], ssem[s]).start()

    def wait_store(s):
        pltpu.make_async_copy(rows[s], out_hbm.at[row_base], ssem[s]).wait()

    def compute(c, s):
        r = rows[s]

        def body(i, carry):
            off = i * L
            ev16 = expr_v[c, pl.ds(off, L)]
            for u in range(L):
                ii = off + u
                ev = jnp.full((L,), ev16[u], dtype=jnp.float32)
                for q in range(4):
                    sl = pl.ds(q * L, L)
                    r[ii, sl] = r[ii, sl] + (ev * wq[q] + bq[q])
            return carry

        lax.fori_loop(0, S // L, body, 0)

        ev16 = expr_v[c, pl.ds(S - L, L)]
        for u in range(L - (S - S // L * L), L):
            ii = S - L + u
            ev = jnp.full((L,), ev16[u], dtype=jnp.float32)
            for q in range(4):
                sl = pl.ds(q * L, L)
                r[ii, sl] = r[ii, sl] + (ev * wq[q] + bq[q])

    def iteration(j, slot, do_wait_store, do_gather):
        s2 = (slot + 2) % NBUF
        if do_wait_store:
            wait_store(s2)
        if do_gather:
            start_gather(j + 2, s2)
        wait_gather(slot)
        compute(j, slot)
        start_store(j, slot)

    start_gather(0, 0)
    start_gather(1, 1)

    iteration(0, 0, False, True)
    iteration(1, 1, False, True)

    def quad(qi, carry):
        jbase = 2 + 4 * qi
        for bpos in range(4):
            iteration(jbase + bpos, (2 + bpos) % NBUF, True, True)
        return carry

    lax.fori_loop(0, (RPW - 4) // NBUF, quad, 0)

    iteration(RPW - 2, (RPW - 2) % NBUF, True, False)
    iteration(RPW - 1, (RPW - 1) % NBUF, True, False)

    wait_store((RPW - 2) % NBUF)
    wait_store((RPW - 1) % NBUF)


def kernel(gene_ids, expr_values, emb_table, proj_w, proj_b):
    idx = gene_ids.astype(jnp.int32)
    expr = expr_values.astype(jnp.float32)
    w = proj_w.reshape(D).astype(jnp.float32)
    b = proj_b.reshape(D).astype(jnp.float32)
    return _sc_tokenize(idx, expr, emb_table.astype(jnp.float32), w, b)
